# baseline (speedup 1.0000x reference)
import jax
import jax.numpy as jnp
from jax import lax
from jax.experimental import pallas as pl
from jax.experimental.pallas import tpu as pltpu
from jax.experimental.pallas import tpu_sc as plsc

_B = 16384
_K = 32
_NC = 2
_NS = 16
_BPS = _B // _NS
_CH = 512
_ROW = 128


def _sc_gather_rows(u128, i128, uq, iq):
    mesh = plsc.VectorSubcoreMesh(core_axis_name="c", subcore_axis_name="s")

    @pl.kernel(
        out_type=(jax.ShapeDtypeStruct((_B, _ROW), jnp.float32),
                  jax.ShapeDtypeStruct((_B, _ROW), jnp.float32)),
        mesh=mesh,
        scratch_types=[
            pltpu.VMEM((_CH,), jnp.int32),
            pltpu.VMEM((_CH, _ROW), jnp.float32),
            pltpu.SemaphoreType.DMA,
        ],
    )
    def gather_kernel(u_hbm, i_hbm, uq_hbm, iq_hbm, uo_hbm, io_hbm,
                      idx_v, rows_v, sem):
        wid = lax.axis_index("s") * _NC + lax.axis_index("c")
        base = wid * _CH

        def chunk(tbl_hbm, q_hbm, o_hbm):
            pltpu.sync_copy(q_hbm.at[pl.ds(base, _CH)], idx_v)
            pltpu.async_copy(tbl_hbm.at[idx_v], rows_v, sem).wait()
            pltpu.sync_copy(rows_v, o_hbm.at[pl.ds(base, _CH)])

        chunk(u_hbm, uq_hbm, uo_hbm)
        chunk(i_hbm, iq_hbm, io_hbm)

    return gather_kernel(u128, i128, uq, iq)


_PBLK = 8192


def _pack_body(in_ref, eye_ref, out_ref):
    out_ref[...] = jax.lax.dot_general(
        in_ref[...], eye_ref[...], (((0,), (0,)), ((), ())),
        preferred_element_type=jnp.float32)


def _pack_table(tbl_t, eye):
    n = tbl_t.shape[1]
    steps = (n + _PBLK - 1) // _PBLK
    return pl.pallas_call(
        _pack_body,
        grid=(steps,),
        in_specs=[pl.BlockSpec((_K, _PBLK), lambda i: (0, i)),
                  pl.BlockSpec((_K, _K), lambda i: (0, 0))],
        out_specs=pl.BlockSpec((_PBLK, _K), lambda i: (i, 0)),
        out_shape=jax.ShapeDtypeStruct((n, _K), jnp.float32),
    )(tbl_t, eye)


def _select32(rows, sub):
    out = jnp.where(sub == 0, rows[:, 0 * _K:1 * _K], 0.0)
    out += jnp.where(sub == 1, rows[:, 1 * _K:2 * _K], 0.0)
    out += jnp.where(sub == 2, rows[:, 2 * _K:3 * _K], 0.0)
    out += jnp.where(sub == 3, rows[:, 3 * _K:4 * _K], 0.0)
    return out


def _mlp_body(ug_ref, ig_ref, su_ref, si_ref, w1u_ref, w1i_ref, b1_ref,
              w2_ref, cvr_ref, ctr_ref, ctcvr_ref):
    ue = _select32(ug_ref[...], su_ref[...])
    ie = _select32(ig_ref[...], si_ref[...])
    h = jnp.dot(ue, w1u_ref[...], preferred_element_type=jnp.float32)
    h += jnp.dot(ie, w1i_ref[...], preferred_element_type=jnp.float32)
    h = jnp.maximum(h + b1_ref[...], 0.0)
    ctr = jnp.sum(h * w2_ref[...], axis=1, keepdims=True)
    cvr = jnp.sum(ue * ie, axis=1, keepdims=True)
    cvr_ref[...] = cvr
    ctr_ref[...] = ctr
    ctcvr_ref[...] = jax.nn.sigmoid(ctr) * jax.nn.sigmoid(cvr)


def kernel(x, user_table, item_table, W1, b1, W2):
    xi = x.astype(jnp.int32)
    user_idx = xi[:, 0]
    item_idx = xi[:, 1]

    n4 = user_table.shape[0] // 4
    eye = jnp.eye(_K, dtype=jnp.float32)
    u128 = _pack_table(user_table.T, eye).reshape(n4, _ROW)
    i128 = _pack_table(item_table.T, eye).reshape(n4, _ROW)

    uq = user_idx >> 2
    iq = item_idx >> 2
    ug, ig = _sc_gather_rows(u128, i128, uq, iq)

    su = (user_idx & 3).reshape(_B, 1)
    si = (item_idx & 3).reshape(_B, 1)

    w1u = W1[:_K]
    w1i = W1[_K:]
    b1r = b1.reshape(1, _K)
    w2r = W2.reshape(1, _K)

    out_t = jax.ShapeDtypeStruct((_B, 1), jnp.float32)
    blk = 2048
    grid = _B // blk
    cvr, ctr, ctcvr = pl.pallas_call(
        _mlp_body,
        grid=(grid,),
        in_specs=[
            pl.BlockSpec((blk, _ROW), lambda i: (i, 0)),
            pl.BlockSpec((blk, _ROW), lambda i: (i, 0)),
            pl.BlockSpec((blk, 1), lambda i: (i, 0)),
            pl.BlockSpec((blk, 1), lambda i: (i, 0)),
            pl.BlockSpec((_K, _K), lambda i: (0, 0)),
            pl.BlockSpec((_K, _K), lambda i: (0, 0)),
            pl.BlockSpec((1, _K), lambda i: (0, 0)),
            pl.BlockSpec((1, _K), lambda i: (0, 0)),
        ],
        out_specs=(
            pl.BlockSpec((blk, 1), lambda i: (i, 0)),
            pl.BlockSpec((blk, 1), lambda i: (i, 0)),
            pl.BlockSpec((blk, 1), lambda i: (i, 0)),
        ),
        out_shape=(out_t, out_t, out_t),
    )(ug, ig, su, si, w1u, w1i, b1r, w2r)
    return (cvr, ctr, ctcvr)


# speedup vs baseline: 1.9164x; 1.9164x over previous
import jax
import jax.numpy as jnp
from jax import lax
from jax.experimental import pallas as pl
from jax.experimental.pallas import tpu as pltpu
from jax.experimental.pallas import tpu_sc as plsc

_B = 16384
_K = 32
_NC = 2
_NS = 16
_BPS = _B // _NS
_CH = 512
_ROW = 128


def _sc_gather_rows(u128, i128, uq, iq):
    mesh = plsc.VectorSubcoreMesh(core_axis_name="c", subcore_axis_name="s")

    @pl.kernel(
        out_type=(jax.ShapeDtypeStruct((_B, _ROW), jnp.float32),
                  jax.ShapeDtypeStruct((_B, _ROW), jnp.float32)),
        mesh=mesh,
        scratch_types=[
            pltpu.VMEM((_CH,), jnp.int32),
            pltpu.VMEM((_CH, _ROW), jnp.float32),
            pltpu.SemaphoreType.DMA,
        ],
    )
    def gather_kernel(u_hbm, i_hbm, uq_hbm, iq_hbm, uo_hbm, io_hbm,
                      idx_v, rows_v, sem):
        wid = lax.axis_index("s") * _NC + lax.axis_index("c")
        base = wid * _CH

        def chunk(tbl_hbm, q_hbm, o_hbm):
            pltpu.sync_copy(q_hbm.at[pl.ds(base, _CH)], idx_v)
            pltpu.async_copy(tbl_hbm.at[idx_v], rows_v, sem).wait()
            pltpu.sync_copy(rows_v, o_hbm.at[pl.ds(base, _CH)])

        chunk(u_hbm, uq_hbm, uo_hbm)
        chunk(i_hbm, iq_hbm, io_hbm)

    return gather_kernel(u128, i128, uq, iq)


_PBLK = 8192


def _pack_body(in_ref, eye_ref, out_ref):
    out_ref[...] = jax.lax.dot_general(
        in_ref[...], eye_ref[...], (((0,), (0,)), ((), ())),
        preferred_element_type=jnp.float32)


def _pack_table(tbl_t, eye):
    n = tbl_t.shape[1]
    steps = (n + _PBLK - 1) // _PBLK
    return pl.pallas_call(
        _pack_body,
        grid=(steps,),
        in_specs=[pl.BlockSpec((_K, _PBLK), lambda i: (0, i)),
                  pl.BlockSpec((_K, _K), lambda i: (0, 0))],
        out_specs=pl.BlockSpec((_PBLK, _K), lambda i: (i, 0)),
        out_shape=jax.ShapeDtypeStruct((n, _K), jnp.float32),
    )(tbl_t, eye)


def _select32(rows, sub):
    out = jnp.where(sub == 0, rows[:, 0 * _K:1 * _K], 0.0)
    out += jnp.where(sub == 1, rows[:, 1 * _K:2 * _K], 0.0)
    out += jnp.where(sub == 2, rows[:, 2 * _K:3 * _K], 0.0)
    out += jnp.where(sub == 3, rows[:, 3 * _K:4 * _K], 0.0)
    return out


def _mlp_body(ug_ref, ig_ref, su_ref, si_ref, w1u_ref, w1i_ref, b1_ref,
              w2_ref, cvr_ref, ctr_ref, ctcvr_ref):
    ue = _select32(ug_ref[...], su_ref[...])
    ie = _select32(ig_ref[...], si_ref[...])
    h = jnp.dot(ue, w1u_ref[...], preferred_element_type=jnp.float32)
    h += jnp.dot(ie, w1i_ref[...], preferred_element_type=jnp.float32)
    h = jnp.maximum(h + b1_ref[...], 0.0)
    ctr = jnp.sum(h * w2_ref[...], axis=1, keepdims=True)
    cvr = jnp.sum(ue * ie, axis=1, keepdims=True)
    cvr_ref[...] = cvr
    ctr_ref[...] = ctr
    ctcvr_ref[...] = jax.nn.sigmoid(ctr) * jax.nn.sigmoid(cvr)


def kernel(x, user_table, item_table, W1, b1, W2):
    xi = x.astype(jnp.int32)
    user_idx = xi[:, 0]
    item_idx = xi[:, 1]

    n4 = user_table.shape[0] // 4
    eye = jnp.eye(_K, dtype=jnp.float32)
    u128 = _pack_table(user_table.T, eye).reshape(n4, _ROW)
    i128 = _pack_table(item_table.T, eye).reshape(n4, _ROW)

    uq = user_idx >> 2
    iq = item_idx >> 2
    ug = u128[:_B]
    ig = i128[:_B]

    su = (user_idx & 3).reshape(_B, 1)
    si = (item_idx & 3).reshape(_B, 1)

    w1u = W1[:_K]
    w1i = W1[_K:]
    b1r = b1.reshape(1, _K)
    w2r = W2.reshape(1, _K)

    out_t = jax.ShapeDtypeStruct((_B, 1), jnp.float32)
    blk = 2048
    grid = _B // blk
    cvr, ctr, ctcvr = pl.pallas_call(
        _mlp_body,
        grid=(grid,),
        in_specs=[
            pl.BlockSpec((blk, _ROW), lambda i: (i, 0)),
            pl.BlockSpec((blk, _ROW), lambda i: (i, 0)),
            pl.BlockSpec((blk, 1), lambda i: (i, 0)),
            pl.BlockSpec((blk, 1), lambda i: (i, 0)),
            pl.BlockSpec((_K, _K), lambda i: (0, 0)),
            pl.BlockSpec((_K, _K), lambda i: (0, 0)),
            pl.BlockSpec((1, _K), lambda i: (0, 0)),
            pl.BlockSpec((1, _K), lambda i: (0, 0)),
        ],
        out_specs=(
            pl.BlockSpec((blk, 1), lambda i: (i, 0)),
            pl.BlockSpec((blk, 1), lambda i: (i, 0)),
            pl.BlockSpec((blk, 1), lambda i: (i, 0)),
        ),
        out_shape=(out_t, out_t, out_t),
    )(ug, ig, su, si, w1u, w1i, b1r, w2r)
    return (cvr, ctr, ctcvr)


# speedup vs baseline: 2.2210x; 1.1589x over previous
import jax
import jax.numpy as jnp
from jax import lax
from jax.experimental import pallas as pl
from jax.experimental.pallas import tpu as pltpu
from jax.experimental.pallas import tpu_sc as plsc

_B = 16384
_K = 32
_NC = 2
_NS = 16
_NW = _NC * _NS
_CH = _B // _NW
_ROW = 128
_PBLK = 16384
_QB = _PBLK // 4


def _sc_gather_rows(u128, i128, uq, iq):
    mesh = plsc.VectorSubcoreMesh(core_axis_name="c", subcore_axis_name="s")

    @pl.kernel(
        out_type=(jax.ShapeDtypeStruct((_B, _ROW), jnp.float32),
                  jax.ShapeDtypeStruct((_B, _ROW), jnp.float32)),
        mesh=mesh,
        scratch_types=[
            pltpu.VMEM((_CH,), jnp.int32),
            pltpu.VMEM((_CH, _ROW), jnp.float32),
            pltpu.SemaphoreType.DMA,
        ],
    )
    def gather_kernel(u_hbm, i_hbm, uq_hbm, iq_hbm, uo_hbm, io_hbm,
                      idx_v, rows_v, sem):
        wid = lax.axis_index("s") * _NC + lax.axis_index("c")
        base = wid * _CH

        def chunk(tbl_hbm, q_hbm, o_hbm):
            pltpu.sync_copy(q_hbm.at[pl.ds(base, _CH)], idx_v)
            pltpu.async_copy(tbl_hbm.at[idx_v], rows_v, sem).wait()
            pltpu.sync_copy(rows_v, o_hbm.at[pl.ds(base, _CH)])

        chunk(u_hbm, uq_hbm, uo_hbm)
        chunk(i_hbm, iq_hbm, io_hbm)

    return gather_kernel(u128, i128, uq, iq)


def _pack_body(in_ref, eye_ref, out_ref):
    x = in_ref[...]
    eye = eye_ref[...]
    quarters = [
        jax.lax.dot_general(
            x[:, c * _QB:(c + 1) * _QB], eye, (((0,), (0,)), ((), ())),
            preferred_element_type=jnp.float32)
        for c in range(4)
    ]
    out_ref[...] = jnp.concatenate(quarters, axis=1)


def _pack_table(tbl_t, eye, steps):
    return pl.pallas_call(
        _pack_body,
        grid=(steps,),
        in_specs=[pl.BlockSpec((_K, _PBLK), lambda i: (0, i)),
                  pl.BlockSpec((_K, _K), lambda i: (0, 0))],
        out_specs=pl.BlockSpec((_QB, _ROW), lambda i: (i, 0)),
        out_shape=jax.ShapeDtypeStruct((steps * _QB, _ROW), jnp.float32),
    )(tbl_t, eye)


def _mlp_body(ug_ref, ig_ref, um_ref, im_ref, w1u_ref, w1i_ref, b1_ref,
              w2_ref, cvr_ref, ctr_ref, ctcvr_ref):
    up = ug_ref[...] * um_ref[...]
    ip = ig_ref[...] * im_ref[...]
    ue = (up[:, 0 * _K:1 * _K] + up[:, 1 * _K:2 * _K]
          + up[:, 2 * _K:3 * _K] + up[:, 3 * _K:4 * _K])
    ie = (ip[:, 0 * _K:1 * _K] + ip[:, 1 * _K:2 * _K]
          + ip[:, 2 * _K:3 * _K] + ip[:, 3 * _K:4 * _K])
    h = jnp.dot(ue, w1u_ref[...], preferred_element_type=jnp.float32)
    h += jnp.dot(ie, w1i_ref[...], preferred_element_type=jnp.float32)
    h = jnp.maximum(h + b1_ref[...], 0.0)
    ctr = jnp.sum(h * w2_ref[...], axis=1, keepdims=True)
    cvr = jnp.sum(ue * ie, axis=1, keepdims=True)
    cvr_ref[...] = cvr
    ctr_ref[...] = ctr
    ctcvr_ref[...] = jax.nn.sigmoid(ctr) * jax.nn.sigmoid(cvr)


def kernel(x, user_table, item_table, W1, b1, W2):
    xi = x.astype(jnp.int32)
    user_idx = xi[:, 0]
    item_idx = xi[:, 1]

    n = user_table.shape[0]
    steps = (n + _PBLK - 1) // _PBLK
    eye = jnp.eye(_K, dtype=jnp.float32)
    u128 = _pack_table(user_table.T, eye, steps)
    i128 = _pack_table(item_table.T, eye, steps)

    uq = ((user_idx >> 14) << 12) | (user_idx & (_QB - 1))
    iq = ((item_idx >> 14) << 12) | (item_idx & (_QB - 1))
    ug, ig = _sc_gather_rows(u128, i128, uq, iq)

    su = (user_idx >> 12) & 3
    si = (item_idx >> 12) & 3
    band = jnp.arange(_ROW, dtype=jnp.int32)[None, :] >> 5
    umask = (band == su[:, None]).astype(jnp.float32)
    imask = (band == si[:, None]).astype(jnp.float32)

    w1u = W1[:_K]
    w1i = W1[_K:]
    b1r = b1.reshape(1, _K)
    w2r = W2.reshape(1, _K)

    out_t = jax.ShapeDtypeStruct((_B, 1), jnp.float32)
    blk = 2048
    grid = _B // blk
    cvr, ctr, ctcvr = pl.pallas_call(
        _mlp_body,
        grid=(grid,),
        in_specs=[
            pl.BlockSpec((blk, _ROW), lambda i: (i, 0)),
            pl.BlockSpec((blk, _ROW), lambda i: (i, 0)),
            pl.BlockSpec((blk, _ROW), lambda i: (i, 0)),
            pl.BlockSpec((blk, _ROW), lambda i: (i, 0)),
            pl.BlockSpec((_K, _K), lambda i: (0, 0)),
            pl.BlockSpec((_K, _K), lambda i: (0, 0)),
            pl.BlockSpec((1, _K), lambda i: (0, 0)),
            pl.BlockSpec((1, _K), lambda i: (0, 0)),
        ],
        out_specs=(
            pl.BlockSpec((blk, 1), lambda i: (i, 0)),
            pl.BlockSpec((blk, 1), lambda i: (i, 0)),
            pl.BlockSpec((blk, 1), lambda i: (i, 0)),
        ),
        out_shape=(out_t, out_t, out_t),
    )(ug, ig, umask, imask, w1u, w1i, b1r, w2r)
    return (cvr, ctr, ctcvr)


# speedup vs baseline: 2.2374x; 1.0074x over previous
import jax
import jax.numpy as jnp
from jax import lax
from jax.experimental import pallas as pl
from jax.experimental.pallas import tpu as pltpu
from jax.experimental.pallas import tpu_sc as plsc

_B = 16384
_K = 32
_NC = 2
_NS = 16
_NW = _NC * _NS
_CH = _B // _NW
_ROW = 128
_PBLK = 32768
_QB = _PBLK // 4


def _sc_gather_rows(u128, i128, uq, iq):
    mesh = plsc.VectorSubcoreMesh(core_axis_name="c", subcore_axis_name="s")

    @pl.kernel(
        out_type=(jax.ShapeDtypeStruct((_B, _ROW), jnp.float32),
                  jax.ShapeDtypeStruct((_B, _ROW), jnp.float32)),
        mesh=mesh,
        scratch_types=[
            pltpu.VMEM((_CH,), jnp.int32),
            pltpu.VMEM((_CH, _ROW), jnp.float32),
            pltpu.SemaphoreType.DMA,
        ],
    )
    def gather_kernel(u_hbm, i_hbm, uq_hbm, iq_hbm, uo_hbm, io_hbm,
                      idx_v, rows_v, sem):
        wid = lax.axis_index("s") * _NC + lax.axis_index("c")
        base = wid * _CH

        def chunk(tbl_hbm, q_hbm, o_hbm):
            pltpu.sync_copy(q_hbm.at[pl.ds(base, _CH)], idx_v)
            pltpu.async_copy(tbl_hbm.at[idx_v], rows_v, sem).wait()
            pltpu.sync_copy(rows_v, o_hbm.at[pl.ds(base, _CH)])

        chunk(u_hbm, uq_hbm, uo_hbm)
        chunk(i_hbm, iq_hbm, io_hbm)

    return gather_kernel(u128, i128, uq, iq)


def _pack_body(in_ref, eye_ref, out_ref):
    x = in_ref[...]
    eye = eye_ref[...]
    quarters = [
        jax.lax.dot_general(
            x[:, c * _QB:(c + 1) * _QB], eye, (((0,), (0,)), ((), ())),
            preferred_element_type=jnp.float32)
        for c in range(4)
    ]
    out_ref[...] = jnp.concatenate(quarters, axis=1)


def _pack_table(tbl_t, eye, steps):
    return pl.pallas_call(
        _pack_body,
        grid=(steps,),
        in_specs=[pl.BlockSpec((_K, _PBLK), lambda i: (0, i)),
                  pl.BlockSpec((_K, _K), lambda i: (0, 0))],
        out_specs=pl.BlockSpec((_QB, _ROW), lambda i: (i, 0)),
        out_shape=jax.ShapeDtypeStruct((steps * _QB, _ROW), jnp.float32),
    )(tbl_t, eye)


def _mlp_body(ug_ref, ig_ref, um_ref, im_ref, w1u_ref, w1i_ref, b1_ref,
              w2_ref, cvr_ref, ctr_ref, ctcvr_ref):
    up = ug_ref[...] * um_ref[...]
    ip = ig_ref[...] * im_ref[...]
    ue = (up[:, 0 * _K:1 * _K] + up[:, 1 * _K:2 * _K]
          + up[:, 2 * _K:3 * _K] + up[:, 3 * _K:4 * _K])
    ie = (ip[:, 0 * _K:1 * _K] + ip[:, 1 * _K:2 * _K]
          + ip[:, 2 * _K:3 * _K] + ip[:, 3 * _K:4 * _K])
    h = jnp.dot(ue, w1u_ref[...], preferred_element_type=jnp.float32)
    h += jnp.dot(ie, w1i_ref[...], preferred_element_type=jnp.float32)
    h = jnp.maximum(h + b1_ref[...], 0.0)
    ctr = jnp.sum(h * w2_ref[...], axis=1, keepdims=True)
    cvr = jnp.sum(ue * ie, axis=1, keepdims=True)
    cvr_ref[...] = cvr
    ctr_ref[...] = ctr
    ctcvr_ref[...] = jax.nn.sigmoid(ctr) * jax.nn.sigmoid(cvr)


def kernel(x, user_table, item_table, W1, b1, W2):
    xi = x.astype(jnp.int32)
    user_idx = xi[:, 0]
    item_idx = xi[:, 1]

    n = user_table.shape[0]
    steps = (n + _PBLK - 1) // _PBLK
    eye = jnp.eye(_K, dtype=jnp.float32)
    u128 = _pack_table(user_table.T, eye, steps)
    i128 = _pack_table(item_table.T, eye, steps)

    sh = _PBLK.bit_length() - 1
    uq = ((user_idx >> sh) << (sh - 2)) | (user_idx & (_QB - 1))
    iq = ((item_idx >> sh) << (sh - 2)) | (item_idx & (_QB - 1))
    ug, ig = _sc_gather_rows(u128, i128, uq, iq)

    su = (user_idx >> (sh - 2)) & 3
    si = (item_idx >> (sh - 2)) & 3
    band = jnp.arange(_ROW, dtype=jnp.int32)[None, :] >> 5
    umask = (band == su[:, None]).astype(jnp.float32)
    imask = (band == si[:, None]).astype(jnp.float32)

    w1u = W1[:_K]
    w1i = W1[_K:]
    b1r = b1.reshape(1, _K)
    w2r = W2.reshape(1, _K)

    out_t = jax.ShapeDtypeStruct((_B, 1), jnp.float32)
    blk = 2048
    grid = _B // blk
    cvr, ctr, ctcvr = pl.pallas_call(
        _mlp_body,
        grid=(grid,),
        in_specs=[
            pl.BlockSpec((blk, _ROW), lambda i: (i, 0)),
            pl.BlockSpec((blk, _ROW), lambda i: (i, 0)),
            pl.BlockSpec((blk, _ROW), lambda i: (i, 0)),
            pl.BlockSpec((blk, _ROW), lambda i: (i, 0)),
            pl.BlockSpec((_K, _K), lambda i: (0, 0)),
            pl.BlockSpec((_K, _K), lambda i: (0, 0)),
            pl.BlockSpec((1, _K), lambda i: (0, 0)),
            pl.BlockSpec((1, _K), lambda i: (0, 0)),
        ],
        out_specs=(
            pl.BlockSpec((blk, 1), lambda i: (i, 0)),
            pl.BlockSpec((blk, 1), lambda i: (i, 0)),
            pl.BlockSpec((blk, 1), lambda i: (i, 0)),
        ),
        out_shape=(out_t, out_t, out_t),
    )(ug, ig, umask, imask, w1u, w1i, b1r, w2r)
    return (cvr, ctr, ctcvr)
